# baseline (speedup 1.0000x reference)
import dataclasses
import functools

import jax
import jax.numpy as jnp
from jax import lax
from jax.experimental import pallas as pl
from jax.experimental.pallas import tpu as pltpu
from jax.experimental.pallas import tpu_sc as plsc

N = 10000
E = 320000
D = 128
NC = 2
NS = 16
NW = NC * NS
LANES = 16
CH = 64
CHUNKS = 160
E_PAD = NW * CHUNKS * CH
N_PAD = 10240
ROWS_PER_TILE = N_PAD // NS
NG = 4
NM = 8


def _bcast16(v, e):
    idx = jnp.full((LANES, 1), e, dtype=jnp.int32)
    dn = lax.GatherDimensionNumbers(
        offset_dims=(), collapsed_slice_dims=(0,), start_index_map=(0,))
    return lax.gather(v, idx, dn, (1,),
                      mode=lax.GatherScatterMode.PROMISE_IN_BOUNDS)


def _sc_body(x_hbm, epack_hbm, zeros_hbm, out_hbm, acc_sh, *rest):
    c = lax.axis_index("c")
    s = lax.axis_index("s")
    wid = c * NS + s
    r0 = s * ROWS_PER_TILE
    m_bufs = rest[0:NM]
    g_bufs = rest[NM:NM + NG]
    msems = rest[NM + NG:2 * NM + NG]
    gsems = rest[2 * NM + NG:2 * NM + 2 * NG]
    ssems = rest[2 * NM + 2 * NG:2 * NM + 3 * NG]
    cbase = wid * CHUNKS

    def start_meta(k, im):
        pltpu.async_copy(epack_hbm.at[cbase + k], m_bufs[im], msems[im])

    def wait_meta(k, im):
        pltpu.make_async_copy(epack_hbm.at[cbase + k], m_bufs[im],
                              msems[im]).wait()

    def start_gather(im, ig):
        pltpu.async_copy(x_hbm.at[m_bufs[im].at[1]], g_bufs[ig], gsems[ig])

    def wait_gather(im, ig):
        pltpu.make_async_copy(x_hbm.at[m_bufs[im].at[1]], g_bufs[ig],
                              gsems[ig]).wait()

    def start_scatter(im, ig):
        pltpu.async_copy(g_bufs[ig], acc_sh.at[m_bufs[im].at[0]],
                         ssems[ig], add=True)

    def wait_scatter(im, ig):
        pltpu.make_async_copy(g_bufs[ig], acc_sh.at[m_bufs[im].at[0]],
                              ssems[ig]).wait()

    for i in range(NM):
        start_meta(i, i)

    pltpu.sync_copy(zeros_hbm.at[pl.ds(r0, ROWS_PER_TILE)],
                    acc_sh.at[pl.ds(r0, ROWS_PER_TILE)])
    plsc.subcore_barrier()

    for i in range(NG):
        wait_meta(i, i)
        start_gather(i, i)

    def _process(k, i):
        ig = i % NG
        g_v = g_bufs[ig]
        m_v = m_bufs[i]
        wait_gather(i, ig)

        @pl.loop(0, CH, step=LANES)
        def _scale(e0):
            vals16 = plsc.bitcast(m_v.at[2][pl.ds(e0, LANES)], jnp.float32)

            @pl.loop(0, LANES)
            def _edge(e):
                bc = _bcast16(vals16, e)
                r = e0 + e
                for f in range(D // LANES):
                    sl = pl.ds(f * LANES, LANES)
                    g_v[r, sl] = g_v[r, sl] * bc

        start_scatter(i, ig)

        ip = (i - 1) % NM
        igp = (i - 1) % NG

        @pl.when(jnp.logical_and(k >= 1, k + NG - 1 < CHUNKS))
        def _():
            wait_scatter(ip, igp)
            @pl.when(k - 1 + NM < CHUNKS)
            def _():
                start_meta(k - 1 + NM, ip)
            im_next = (i + NG - 1) % NM
            wait_meta(k + NG - 1, im_next)
            start_gather(im_next, igp)

    @pl.loop(0, CHUNKS, step=NM)
    def _chunk(j):
        for i in range(NM):
            _process(j + i, i)

    for k in range(CHUNKS - NG, CHUNKS):
        wait_scatter(k % NM, k % NG)
    plsc.subcore_barrier()
    pltpu.sync_copy(acc_sh.at[pl.ds(r0, ROWS_PER_TILE)],
                    out_hbm.at[pl.ds(c * N_PAD + r0, ROWS_PER_TILE)])


def _sc_aggregate(x, epack, zeros):
    mesh = plsc.VectorSubcoreMesh(core_axis_name="c", subcore_axis_name="s")
    cp = pltpu.CompilerParams()
    if "needs_layout_passes" in pltpu.CompilerParams.__dataclass_fields__:
        cp = dataclasses.replace(cp, needs_layout_passes=False)
    scratch = [pltpu.VMEM_SHARED((N_PAD, D), jnp.float32)]
    scratch += [pltpu.VMEM((3, CH), jnp.int32) for _ in range(NM)]
    scratch += [pltpu.VMEM((CH, D), jnp.float32) for _ in range(NG)]
    scratch += [pltpu.SemaphoreType.DMA for _ in range(NM + 2 * NG)]
    kern = pl.kernel(
        _sc_body,
        out_type=jax.ShapeDtypeStruct((NC * N_PAD, D), jnp.float32),
        mesh=mesh,
        scratch_types=scratch,
        compiler_params=cp,
    )
    return kern(x, epack, zeros)


def _tc_combine(partials, W):
    p3 = partials.reshape(NC, N_PAD, D)
    BR = 2000

    def body(p_ref, w_ref, o_ref):
        ssum = p_ref[0] + p_ref[1]
        y = jnp.dot(ssum, w_ref[...], preferred_element_type=jnp.float32,
                    precision=lax.Precision.HIGHEST)
        o_ref[...] = jnp.maximum(y, 0.0)

    return pl.pallas_call(
        body,
        grid=(N // BR,),
        in_specs=[pl.BlockSpec((NC, BR, D), lambda i: (0, i, 0)),
                  pl.BlockSpec((D, D), lambda i: (0, 0))],
        out_specs=pl.BlockSpec((BR, D), lambda i: (i, 0)),
        out_shape=jax.ShapeDtypeStruct((N, D), jnp.float32),
    )(p3, W)


def kernel(x, sup_indices, sup_values, W):
    rows = sup_indices[0].astype(jnp.int32)
    cols = sup_indices[1].astype(jnp.int32)
    vals = sup_values.astype(jnp.float32)
    pad = E_PAD - E
    spread = jnp.arange(pad, dtype=jnp.int32) % N
    rows = jnp.concatenate([rows, spread])
    cols = jnp.concatenate([cols, spread])
    vals = jnp.concatenate([vals, jnp.zeros((pad,), jnp.float32)])
    vbits = lax.bitcast_convert_type(vals, jnp.int32)
    epack = jnp.stack([rows, cols, vbits], axis=0)
    epack = epack.reshape(3, E_PAD // CH, CH).transpose(1, 0, 2)
    zeros = jnp.zeros((N_PAD, D), jnp.float32)
    partials = _sc_aggregate(x, epack, zeros)
    return _tc_combine(partials, W)


# speedup vs baseline: 1.0012x; 1.0012x over previous
import dataclasses
import functools

import jax
import jax.numpy as jnp
from jax import lax
from jax.experimental import pallas as pl
from jax.experimental.pallas import tpu as pltpu
from jax.experimental.pallas import tpu_sc as plsc

N = 10000
E = 320000
D = 128
NC = 2
NS = 16
NW = NC * NS
LANES = 16
CH = 64
CHUNKS = 160
E_PAD = NW * CHUNKS * CH
N_PAD = 10240
ROWS_PER_TILE = N_PAD // NS
NG = 4
NM = 8


def _bcast16(v, e):
    idx = jnp.full((LANES, 1), e, dtype=jnp.int32)
    dn = lax.GatherDimensionNumbers(
        offset_dims=(), collapsed_slice_dims=(0,), start_index_map=(0,))
    return lax.gather(v, idx, dn, (1,),
                      mode=lax.GatherScatterMode.PROMISE_IN_BOUNDS)


def _sc_body(x_hbm, epack_hbm, zeros_hbm, out_hbm, acc_sh, *rest):
    c = lax.axis_index("c")
    s = lax.axis_index("s")
    wid = c * NS + s
    r0 = s * ROWS_PER_TILE
    m_bufs = rest[0:NM]
    g_bufs = rest[NM:NM + NG]
    msems = rest[NM + NG:2 * NM + NG]
    gsems = rest[2 * NM + NG:2 * NM + 3 * NG]
    ssems = rest[2 * NM + 3 * NG:2 * NM + 4 * NG]
    cbase = wid * CHUNKS
    H = CH // 2

    def start_meta(k, im):
        pltpu.async_copy(epack_hbm.at[cbase + k], m_bufs[im], msems[im])

    def wait_meta(k, im):
        pltpu.make_async_copy(epack_hbm.at[cbase + k], m_bufs[im],
                              msems[im]).wait()

    def start_gather(im, ig):
        m = m_bufs[im]
        g = g_bufs[ig]
        pltpu.async_copy(x_hbm.at[m.at[1, pl.ds(0, H)]], g.at[pl.ds(0, H)],
                         gsems[2 * ig])
        pltpu.async_copy(x_hbm.at[m.at[1, pl.ds(H, H)]], g.at[pl.ds(H, H)],
                         gsems[2 * ig + 1])

    def wait_gather(im, ig):
        m = m_bufs[im]
        g = g_bufs[ig]
        pltpu.make_async_copy(x_hbm.at[m.at[1, pl.ds(0, H)]],
                              g.at[pl.ds(0, H)], gsems[2 * ig]).wait()
        pltpu.make_async_copy(x_hbm.at[m.at[1, pl.ds(H, H)]],
                              g.at[pl.ds(H, H)], gsems[2 * ig + 1]).wait()

    def start_scatter(im, ig):
        pltpu.async_copy(g_bufs[ig], acc_sh.at[m_bufs[im].at[0]],
                         ssems[ig], add=True)

    def wait_scatter(im, ig):
        pltpu.make_async_copy(g_bufs[ig], acc_sh.at[m_bufs[im].at[0]],
                              ssems[ig]).wait()

    for i in range(NM):
        start_meta(i, i)

    pltpu.sync_copy(zeros_hbm.at[pl.ds(r0, ROWS_PER_TILE)],
                    acc_sh.at[pl.ds(r0, ROWS_PER_TILE)])
    plsc.subcore_barrier()

    for i in range(NG):
        wait_meta(i, i)
        start_gather(i, i)

    def _process(k, i):
        ig = i % NG
        g_v = g_bufs[ig]
        m_v = m_bufs[i]
        wait_gather(i, ig)

        @pl.loop(0, CH, step=LANES)
        def _scale(e0):
            vals16 = plsc.bitcast(m_v.at[2][pl.ds(e0, LANES)], jnp.float32)

            @pl.loop(0, LANES)
            def _edge(e):
                bc = _bcast16(vals16, e)
                r = e0 + e
                for f in range(D // LANES):
                    sl = pl.ds(f * LANES, LANES)
                    g_v[r, sl] = g_v[r, sl] * bc

        start_scatter(i, ig)

        ip = (i - 1) % NM
        igp = (i - 1) % NG

        @pl.when(jnp.logical_and(k >= 1, k + NG - 1 < CHUNKS))
        def _():
            wait_scatter(ip, igp)
            @pl.when(k - 1 + NM < CHUNKS)
            def _():
                start_meta(k - 1 + NM, ip)
            im_next = (i + NG - 1) % NM
            wait_meta(k + NG - 1, im_next)
            start_gather(im_next, igp)

    @pl.loop(0, CHUNKS, step=NM)
    def _chunk(j):
        for i in range(NM):
            _process(j + i, i)

    for k in range(CHUNKS - NG, CHUNKS):
        wait_scatter(k % NM, k % NG)
    plsc.subcore_barrier()
    pltpu.sync_copy(acc_sh.at[pl.ds(r0, ROWS_PER_TILE)],
                    out_hbm.at[pl.ds(c * N_PAD + r0, ROWS_PER_TILE)])


def _sc_aggregate(x, epack, zeros):
    mesh = plsc.VectorSubcoreMesh(core_axis_name="c", subcore_axis_name="s")
    cp = pltpu.CompilerParams()
    if "needs_layout_passes" in pltpu.CompilerParams.__dataclass_fields__:
        cp = dataclasses.replace(cp, needs_layout_passes=False)
    scratch = [pltpu.VMEM_SHARED((N_PAD, D), jnp.float32)]
    scratch += [pltpu.VMEM((3, CH), jnp.int32) for _ in range(NM)]
    scratch += [pltpu.VMEM((CH, D), jnp.float32) for _ in range(NG)]
    scratch += [pltpu.SemaphoreType.DMA for _ in range(NM + 3 * NG)]
    kern = pl.kernel(
        _sc_body,
        out_type=jax.ShapeDtypeStruct((NC * N_PAD, D), jnp.float32),
        mesh=mesh,
        scratch_types=scratch,
        compiler_params=cp,
    )
    return kern(x, epack, zeros)


def _tc_combine(partials, W):
    p3 = partials.reshape(NC, N_PAD, D)
    BR = 2000

    def body(p_ref, w_ref, o_ref):
        ssum = p_ref[0] + p_ref[1]
        y = jnp.dot(ssum, w_ref[...], preferred_element_type=jnp.float32,
                    precision=lax.Precision.HIGHEST)
        o_ref[...] = jnp.maximum(y, 0.0)

    return pl.pallas_call(
        body,
        grid=(N // BR,),
        in_specs=[pl.BlockSpec((NC, BR, D), lambda i: (0, i, 0)),
                  pl.BlockSpec((D, D), lambda i: (0, 0))],
        out_specs=pl.BlockSpec((BR, D), lambda i: (i, 0)),
        out_shape=jax.ShapeDtypeStruct((N, D), jnp.float32),
    )(p3, W)


def kernel(x, sup_indices, sup_values, W):
    rows = sup_indices[0].astype(jnp.int32)
    cols = sup_indices[1].astype(jnp.int32)
    vals = sup_values.astype(jnp.float32)
    pad = E_PAD - E
    spread = jnp.arange(pad, dtype=jnp.int32) % N
    rows = jnp.concatenate([rows, spread])
    cols = jnp.concatenate([cols, spread])
    vals = jnp.concatenate([vals, jnp.zeros((pad,), jnp.float32)])
    vbits = lax.bitcast_convert_type(vals, jnp.int32)
    epack = jnp.stack([rows, cols, vbits], axis=0)
    epack = epack.reshape(3, E_PAD // CH, CH).transpose(1, 0, 2)
    zeros = jnp.zeros((N_PAD, D), jnp.float32)
    partials = _sc_aggregate(x, epack, zeros)
    return _tc_combine(partials, W)


# speedup vs baseline: 1.0377x; 1.0365x over previous
import dataclasses
import functools

import jax
import jax.numpy as jnp
from jax import lax
from jax.experimental import pallas as pl
from jax.experimental.pallas import tpu as pltpu
from jax.experimental.pallas import tpu_sc as plsc

N = 10000
E = 320000
D = 128
NC = 2
NS = 16
NW = NC * NS
LANES = 16
CH = 64
CHUNKS = 160
E_PAD = NW * CHUNKS * CH
N_PAD = 10240
ROWS_PER_TILE = N_PAD // NS
NG = 4
NM = 8


def _bcast16(v, e):
    idx = jnp.full((LANES, 1), e, dtype=jnp.int32)
    dn = lax.GatherDimensionNumbers(
        offset_dims=(), collapsed_slice_dims=(0,), start_index_map=(0,))
    return lax.gather(v, idx, dn, (1,),
                      mode=lax.GatherScatterMode.PROMISE_IN_BOUNDS)


def _sc_body(x_hbm, epack_hbm, zeros_hbm, out_hbm, acc_sh, *rest):
    c = lax.axis_index("c")
    s = lax.axis_index("s")
    wid = c * NS + s
    r0 = s * ROWS_PER_TILE
    m_bufs = rest[0:NM]
    g_bufs = rest[NM:NM + NG]
    msems = rest[NM + NG:2 * NM + NG]
    gsems = rest[2 * NM + NG:2 * NM + 3 * NG]
    ssems = rest[2 * NM + 3 * NG:2 * NM + 4 * NG]
    cbase = wid * CHUNKS
    H = CH // 2

    def start_meta(k, im):
        pltpu.async_copy(epack_hbm.at[:, cbase + k], m_bufs[im], msems[im])

    def wait_meta(k, im):
        pltpu.make_async_copy(epack_hbm.at[:, cbase + k], m_bufs[im],
                              msems[im]).wait()

    def start_gather(im, ig):
        m = m_bufs[im]
        g = g_bufs[ig]
        pltpu.async_copy(x_hbm.at[m.at[1, pl.ds(0, H)]], g.at[pl.ds(0, H)],
                         gsems[2 * ig])
        pltpu.async_copy(x_hbm.at[m.at[1, pl.ds(H, H)]], g.at[pl.ds(H, H)],
                         gsems[2 * ig + 1])

    def wait_gather(im, ig):
        m = m_bufs[im]
        g = g_bufs[ig]
        pltpu.make_async_copy(x_hbm.at[m.at[1, pl.ds(0, H)]],
                              g.at[pl.ds(0, H)], gsems[2 * ig]).wait()
        pltpu.make_async_copy(x_hbm.at[m.at[1, pl.ds(H, H)]],
                              g.at[pl.ds(H, H)], gsems[2 * ig + 1]).wait()

    def start_scatter(im, ig):
        pltpu.async_copy(g_bufs[ig], acc_sh.at[m_bufs[im].at[0]],
                         ssems[ig], add=True)

    def wait_scatter(im, ig):
        pltpu.make_async_copy(g_bufs[ig], acc_sh.at[m_bufs[im].at[0]],
                              ssems[ig]).wait()

    for i in range(NM):
        start_meta(i, i)

    pltpu.sync_copy(zeros_hbm.at[pl.ds(r0, ROWS_PER_TILE)],
                    acc_sh.at[pl.ds(r0, ROWS_PER_TILE)])
    plsc.subcore_barrier()

    for i in range(NG):
        wait_meta(i, i)
        start_gather(i, i)

    def _process(k, i):
        ig = i % NG
        g_v = g_bufs[ig]
        m_v = m_bufs[i]
        wait_gather(i, ig)

        @pl.loop(0, CH, step=LANES)
        def _scale(e0):
            vals16 = plsc.bitcast(m_v.at[2][pl.ds(e0, LANES)], jnp.float32)

            @pl.loop(0, LANES)
            def _edge(e):
                bc = _bcast16(vals16, e)
                r = e0 + e
                for f in range(D // LANES):
                    sl = pl.ds(f * LANES, LANES)
                    g_v[r, sl] = g_v[r, sl] * bc

        start_scatter(i, ig)

        ip = (i - 1) % NM
        igp = (i - 1) % NG

        @pl.when(jnp.logical_and(k >= 1, k + NG - 1 < CHUNKS))
        def _():
            wait_scatter(ip, igp)
            @pl.when(k - 1 + NM < CHUNKS)
            def _():
                start_meta(k - 1 + NM, ip)
            im_next = (i + NG - 1) % NM
            wait_meta(k + NG - 1, im_next)
            start_gather(im_next, igp)

    @pl.loop(0, CHUNKS, step=NM)
    def _chunk(j):
        for i in range(NM):
            _process(j + i, i)

    for k in range(CHUNKS - NG, CHUNKS):
        wait_scatter(k % NM, k % NG)
    plsc.subcore_barrier()
    pltpu.sync_copy(acc_sh.at[pl.ds(r0, ROWS_PER_TILE)],
                    out_hbm.at[pl.ds(c * N_PAD + r0, ROWS_PER_TILE)])


def _sc_aggregate(x, epack, zeros):
    mesh = plsc.VectorSubcoreMesh(core_axis_name="c", subcore_axis_name="s")
    cp = pltpu.CompilerParams()
    if "needs_layout_passes" in pltpu.CompilerParams.__dataclass_fields__:
        cp = dataclasses.replace(cp, needs_layout_passes=False)
    scratch = [pltpu.VMEM_SHARED((N_PAD, D), jnp.float32)]
    scratch += [pltpu.VMEM((3, CH), jnp.int32) for _ in range(NM)]
    scratch += [pltpu.VMEM((CH, D), jnp.float32) for _ in range(NG)]
    scratch += [pltpu.SemaphoreType.DMA for _ in range(NM + 3 * NG)]
    kern = pl.kernel(
        _sc_body,
        out_type=jax.ShapeDtypeStruct((NC * N_PAD, D), jnp.float32),
        mesh=mesh,
        scratch_types=scratch,
        compiler_params=cp,
    )
    return kern(x, epack, zeros)


def _tc_combine(partials, W):
    p3 = partials.reshape(NC, N_PAD, D)
    BR = 2000

    def body(p_ref, w_ref, o_ref):
        ssum = p_ref[0] + p_ref[1]
        y = jnp.dot(ssum, w_ref[...], preferred_element_type=jnp.float32,
                    precision=lax.Precision.HIGHEST)
        o_ref[...] = jnp.maximum(y, 0.0)

    return pl.pallas_call(
        body,
        grid=(N // BR,),
        in_specs=[pl.BlockSpec((NC, BR, D), lambda i: (0, i, 0)),
                  pl.BlockSpec((D, D), lambda i: (0, 0))],
        out_specs=pl.BlockSpec((BR, D), lambda i: (i, 0)),
        out_shape=jax.ShapeDtypeStruct((N, D), jnp.float32),
    )(p3, W)


def kernel(x, sup_indices, sup_values, W):
    rows = sup_indices[0].astype(jnp.int32)
    cols = sup_indices[1].astype(jnp.int32)
    vals = sup_values.astype(jnp.float32)
    pad = E_PAD - E
    spread = jnp.arange(pad, dtype=jnp.int32) % N
    rows = jnp.concatenate([rows, spread])
    cols = jnp.concatenate([cols, spread])
    vals = jnp.concatenate([vals, jnp.zeros((pad,), jnp.float32)])
    vbits = lax.bitcast_convert_type(vals, jnp.int32)
    epack = jnp.stack([rows, cols, vbits], axis=0)
    epack = epack.reshape(3, E_PAD // CH, CH)
    zeros = jnp.zeros((N_PAD, D), jnp.float32)
    partials = _sc_aggregate(x, epack, zeros)
    return _tc_combine(partials, W)
